# Initial kernel scaffold; baseline (speedup 1.0000x reference)
import jax
import jax.numpy as jnp
from jax.experimental import pallas as pl


def kernel(x, edge_index, c1_W1, c1_b1, c1_g1, c1_be1, c1_W2, c1_b2, c2_W1, c2_b1, c2_g1, c2_be1, c2_W2, c2_b2, e_W1, e_b1, e_g1, e_be1, e_W2, e_b2):
    raise NotImplementedError("write your pallas kernel here")


# speedup vs baseline: 4.7195x; 4.7195x over previous
import functools

import jax
import jax.numpy as jnp
from jax.experimental import pallas as pl
from jax.experimental.pallas import tpu as pltpu

N = 512
DIN = 512
DIM = 128
E = 16384
TE = 512
NB = E // TE
BI = 8


def _count_kernel(src_ref, dst_ref, c_ref):
    i = pl.program_id(0)

    @pl.when(i == 0)
    def _():
        c_ref[...] = jnp.zeros_like(c_ref)

    src = src_ref[...]
    dst = dst_ref[...]
    cols = jax.lax.broadcasted_iota(jnp.int32, (TE, N), 1)
    oh_s = (src == cols).astype(jnp.bfloat16)
    oh_d = (dst == cols).astype(jnp.bfloat16)
    c_ref[...] += jax.lax.dot_general(
        oh_d, oh_s, (((0,), (0,)), ((), ())),
        preferred_element_type=jnp.float32)


def _ln(t, g, b, eps=1e-5):
    mu = jnp.mean(t, axis=-1, keepdims=True)
    d = t - mu
    var = jnp.mean(d * d, axis=-1, keepdims=True)
    return d * jax.lax.rsqrt(var + eps) * g + b


def _gelu(t):
    return 0.5 * t * (1.0 + jax.lax.erf(t * 0.7071067811865476))


def _mlp_kernel(c_ref, x_ref,
                w1a_ref, b1a_ref, g1a_ref, be1a_ref, w2a_ref, b2a_ref,
                w1b_ref, b1b_ref, g1b_ref, be1b_ref, w2b_ref, b2b_ref,
                ew1_ref, eb1_ref,
                a_ref, b_out_ref):
    C = c_ref[...]
    x = x_ref[...]

    def gin(h, w1, b1, g1, be1, w2, b2):
        t = jnp.dot(h, w1, preferred_element_type=jnp.float32) + b1
        t = _gelu(_ln(t, g1, be1))
        return jnp.dot(t, w2, preferred_element_type=jnp.float32) + b2

    agg1 = jnp.dot(C, x, preferred_element_type=jnp.float32)
    h = gin(x + agg1, w1a_ref[...], b1a_ref[...], g1a_ref[...],
            be1a_ref[...], w2a_ref[...], b2a_ref[...])
    agg2 = jnp.dot(C, h, preferred_element_type=jnp.float32)
    h = gin(h + agg2, w1b_ref[...], b1b_ref[...], g1b_ref[...],
            be1b_ref[...], w2b_ref[...], b2b_ref[...])
    ew1 = ew1_ref[...]
    a_ref[...] = (jnp.dot(h, ew1[:DIM, :], preferred_element_type=jnp.float32)
                  + eb1_ref[...])
    b_out_ref[...] = jnp.dot(h, ew1[DIM:, :], preferred_element_type=jnp.float32)


def _pair_kernel(a_ref, b_ref, g_ref, be_ref, w2_ref, b2_ref, o_ref):
    A = a_ref[...]
    B = b_ref[...]
    z = A[:, None, :] + B[None, :, :]
    mu = jnp.mean(z, axis=-1, keepdims=True)
    d = z - mu
    var = jnp.mean(d * d, axis=-1, keepdims=True)
    y = d * jax.lax.rsqrt(var + 1e-5) * g_ref[...][None] + be_ref[...][None]
    y = _gelu(y)
    o = jnp.sum(y * w2_ref[...][None], axis=-1) + b2_ref[...]
    o_ref[...] = jax.nn.sigmoid(o)


def kernel(x, edge_index, c1_W1, c1_b1, c1_g1, c1_be1, c1_W2, c1_b2,
           c2_W1, c2_b1, c2_g1, c2_be1, c2_W2, c2_b2,
           e_W1, e_b1, e_g1, e_be1, e_W2, e_b2):
    ei = edge_index.astype(jnp.int32)
    src = ei[0].reshape(E, 1)
    dst = ei[1].reshape(E, 1)

    C = pl.pallas_call(
        _count_kernel,
        grid=(NB,),
        in_specs=[pl.BlockSpec((TE, 1), lambda i: (i, 0)),
                  pl.BlockSpec((TE, 1), lambda i: (i, 0))],
        out_specs=pl.BlockSpec((N, N), lambda i: (0, 0)),
        out_shape=jax.ShapeDtypeStruct((N, N), jnp.float32),
    )(src, dst)

    r = lambda v: v.reshape(1, -1)
    A, B = pl.pallas_call(
        _mlp_kernel,
        out_shape=(jax.ShapeDtypeStruct((N, DIM), jnp.float32),
                   jax.ShapeDtypeStruct((N, DIM), jnp.float32)),
    )(C, x,
      c1_W1, r(c1_b1), r(c1_g1), r(c1_be1), c1_W2, r(c1_b2),
      c2_W1, r(c2_b1), r(c2_g1), r(c2_be1), c2_W2, r(c2_b2),
      e_W1, r(e_b1))

    out = pl.pallas_call(
        _pair_kernel,
        grid=(N // BI,),
        in_specs=[pl.BlockSpec((BI, DIM), lambda i: (i, 0)),
                  pl.BlockSpec((N, DIM), lambda i: (0, 0)),
                  pl.BlockSpec((1, DIM), lambda i: (0, 0)),
                  pl.BlockSpec((1, DIM), lambda i: (0, 0)),
                  pl.BlockSpec((1, DIM), lambda i: (0, 0)),
                  pl.BlockSpec((1, 1), lambda i: (0, 0))],
        out_specs=pl.BlockSpec((BI, N), lambda i: (i, 0)),
        out_shape=jax.ShapeDtypeStruct((N, N), jnp.float32),
    )(A, B, r(e_g1), r(e_be1), e_W2.reshape(1, DIM), e_b2.reshape(1, 1))
    return out
